# baseline (speedup 1.0000x reference)
import functools

import jax
import jax.numpy as jnp
from jax import lax
from jax.experimental import pallas as pl
from jax.experimental.pallas import tpu as pltpu
from jax.experimental.pallas import tpu_sc as plsc

_H = 64
_NC = 2
_NS = 16
_NW = _NC * _NS
_CHUNK = 128
_KFIRE = 8


def _tc_head_table(table, w, b2):
    v = table.shape[0]
    blk = 32768
    grid = ((v + blk - 1) // blk,)

    def body(t_ref, w_ref, b_ref, o_ref):
        s = jnp.tanh(0.5 * t_ref[...])
        y = lax.dot_general(
            w_ref[...], s, (((1,), (1,)), ((), ())),
            preferred_element_type=jnp.float32,
        )
        o_ref[...] = y + b_ref[...]

    return pl.pallas_call(
        body,
        grid=grid,
        in_specs=[
            pl.BlockSpec((blk, _H), lambda i: (i, 0)),
            pl.BlockSpec((2, _H), lambda i: (0, 0)),
            pl.BlockSpec((2, 1), lambda i: (0, 0)),
        ],
        out_specs=pl.BlockSpec((2, blk), lambda i: (0, i)),
        out_shape=jax.ShapeDtypeStruct((2, v), jnp.float32),
    )(table, w, b2)


def _sc_lookup(flat, idx3):
    nw, n_chunks, chunk = idx3.shape
    n = nw * n_chunks * chunk
    n_super = n_chunks // _KFIRE
    sup = _KFIRE * chunk
    mesh = plsc.VectorSubcoreMesh(core_axis_name="c", subcore_axis_name="s")

    @functools.partial(
        pl.kernel,
        out_type=jax.ShapeDtypeStruct((n,), jnp.float32),
        mesh=mesh,
        compiler_params=pltpu.CompilerParams(use_tc_tiling_on_sc=False),
        scratch_types=[
            pltpu.VMEM((n_chunks, chunk), jnp.int32),
            pltpu.VMEM((sup,), jnp.float32),
            pltpu.SemaphoreType.DMA,
        ],
    )
    def k(flat_hbm, idx_hbm, out_hbm, idx_v, buf_v, gsem):
        wid = lax.axis_index("s") * _NC + lax.axis_index("c")
        pltpu.sync_copy(idx_hbm.at[wid], idx_v)

        def body(sb, carry):
            copies = []
            for bq in range(_KFIRE):
                j = sb * _KFIRE + bq
                copies.append(pltpu.async_copy(
                    flat_hbm.at[idx_v.at[j]],
                    buf_v.at[pl.ds(bq * chunk, chunk)], gsem))
            for c in copies:
                c.wait()
            base = (wid * n_super + sb) * sup
            pltpu.sync_copy(buf_v, out_hbm.at[pl.ds(base, sup)])
            return carry

        lax.fori_loop(0, n_super, body, 0)

    return k(flat, idx3)


def kernel(x, table, W, b):
    bsz, fields = x.shape
    v = table.shape[0]
    n = bsz * fields
    xf = x.reshape(n, 1)
    xe = jnp.concatenate([xf, xf + v], axis=1)
    n_chunks = (2 * n) // (_NW * _CHUNK)
    idx3 = xe.reshape(_NW, n_chunks, _CHUNK)

    w = 0.5 * W
    b2 = (b + 0.5 * W.sum(axis=1)).reshape(2, 1)

    planes = _tc_head_table(table, w, b2)
    out = _sc_lookup(planes.reshape(2 * v), idx3)
    return out.reshape(bsz, fields, 2)


# speedup vs baseline: 1.2871x; 1.2871x over previous
import functools

import jax
import jax.numpy as jnp
from jax import lax
from jax.experimental import pallas as pl
from jax.experimental.pallas import tpu as pltpu
from jax.experimental.pallas import tpu_sc as plsc

_H = 64
_NC = 2
_NS = 16
_NW = _NC * _NS
_CHUNK = 128
_KFIRE = 16


def _tc_head_table(table, w, b2):
    v = table.shape[0]
    blk = 32768
    grid = ((v + blk - 1) // blk,)

    def body(t_ref, w_ref, b_ref, o_ref):
        s = jnp.tanh(0.5 * t_ref[...])
        y = lax.dot_general(
            w_ref[...], s, (((1,), (1,)), ((), ())),
            preferred_element_type=jnp.float32,
        )
        o_ref[...] = y + b_ref[...]

    return pl.pallas_call(
        body,
        grid=grid,
        in_specs=[
            pl.BlockSpec((blk, _H), lambda i: (i, 0)),
            pl.BlockSpec((2, _H), lambda i: (0, 0)),
            pl.BlockSpec((2, 1), lambda i: (0, 0)),
        ],
        out_specs=pl.BlockSpec((2, blk), lambda i: (0, i)),
        out_shape=jax.ShapeDtypeStruct((2, v), jnp.float32),
    )(table, w, b2)


def _sc_lookup(flat, idx3):
    nw, n_chunks, chunk = idx3.shape
    n = nw * n_chunks * chunk
    n_super = n_chunks // _KFIRE
    sup = _KFIRE * chunk
    mesh = plsc.VectorSubcoreMesh(core_axis_name="c", subcore_axis_name="s")

    @functools.partial(
        pl.kernel,
        out_type=jax.ShapeDtypeStruct((n,), jnp.float32),
        mesh=mesh,
        compiler_params=pltpu.CompilerParams(use_tc_tiling_on_sc=False),
        scratch_types=[
            pltpu.VMEM((n_chunks, chunk), jnp.int32),
            pltpu.VMEM((sup,), jnp.float32),
            pltpu.SemaphoreType.DMA,
        ],
    )
    def k(flat_hbm, idx_hbm, out_hbm, idx_v, buf_v, gsem):
        wid = lax.axis_index("s") * _NC + lax.axis_index("c")
        pltpu.sync_copy(idx_hbm.at[wid], idx_v)

        def body(sb, carry):
            copies = []
            for bq in range(_KFIRE):
                j = sb * _KFIRE + bq
                copies.append(pltpu.async_copy(
                    flat_hbm.at[idx_v.at[j]],
                    buf_v.at[pl.ds(bq * chunk, chunk)], gsem))
            for c in copies:
                c.wait()
            base = (wid * n_super + sb) * sup
            pltpu.sync_copy(buf_v, out_hbm.at[pl.ds(base, sup)])
            return carry

        lax.fori_loop(0, n_super, body, 0)

    return k(flat, idx3)


def kernel(x, table, W, b):
    bsz, fields = x.shape
    v = table.shape[0]
    n = bsz * fields
    x64 = x.reshape(n // _H, _H)
    xe = jnp.repeat(x64, 2, axis=1) + jnp.tile(
        jnp.array([0, v], jnp.int32), _H)
    n_chunks = (2 * n) // (_NW * _CHUNK)
    idx3 = xe.reshape(_NW, n_chunks, _CHUNK)

    w = 0.5 * W
    b2 = (b + 0.5 * W.sum(axis=1)).reshape(2, 1)

    planes = _tc_head_table(table, w, b2)
    out = _sc_lookup(planes.reshape(2 * v), idx3)
    return out.reshape(bsz, fields, 2)


# speedup vs baseline: 1.6851x; 1.3092x over previous
import functools

import jax
import jax.numpy as jnp
from jax import lax
from jax.experimental import pallas as pl
from jax.experimental.pallas import tpu as pltpu
from jax.experimental.pallas import tpu_sc as plsc

_H = 64
_NC = 2
_NS = 16
_NW = _NC * _NS
_CHUNK = 128
_KFIRE = 8


def _tc_head_table(table, w, b2):
    v = table.shape[0]
    blk = 32768
    grid = ((v + blk - 1) // blk,)

    def body(t_ref, w_ref, b_ref, o_ref):
        s = jnp.tanh(0.5 * t_ref[...])
        y = lax.dot_general(
            w_ref[...], s, (((1,), (1,)), ((), ())),
            preferred_element_type=jnp.float32,
        )
        o_ref[...] = y + b_ref[...]

    return pl.pallas_call(
        body,
        grid=grid,
        in_specs=[
            pl.BlockSpec((blk, _H), lambda i: (i, 0)),
            pl.BlockSpec((2, _H), lambda i: (0, 0)),
            pl.BlockSpec((2, 1), lambda i: (0, 0)),
        ],
        out_specs=pl.BlockSpec((2, blk), lambda i: (0, i)),
        out_shape=jax.ShapeDtypeStruct((2, v), jnp.float32),
    )(table, w, b2)


def _sc_lookup(flat, idx_lo, idx_hi):
    nw, n_chunks, chunk = idx_lo.shape
    n = nw * n_chunks * chunk
    n_super = n_chunks // _KFIRE
    sup = _KFIRE * chunk
    mesh = plsc.VectorSubcoreMesh(core_axis_name="c", subcore_axis_name="s")

    @functools.partial(
        pl.kernel,
        out_type=(
            jax.ShapeDtypeStruct((n,), jnp.float32),
            jax.ShapeDtypeStruct((n,), jnp.float32),
        ),
        mesh=mesh,
        compiler_params=pltpu.CompilerParams(use_tc_tiling_on_sc=False),
        scratch_types=[
            pltpu.VMEM((n_chunks, chunk), jnp.int32),
            pltpu.VMEM((n_chunks, chunk), jnp.int32),
            pltpu.VMEM((sup,), jnp.float32),
            pltpu.VMEM((sup,), jnp.float32),
            pltpu.SemaphoreType.DMA,
        ],
    )
    def k(flat_hbm, lo_hbm, hi_hbm, out0_hbm, out1_hbm,
          lo_v, hi_v, buf0_v, buf1_v, gsem):
        wid = lax.axis_index("s") * _NC + lax.axis_index("c")
        pltpu.sync_copy(lo_hbm.at[wid], lo_v)
        pltpu.sync_copy(hi_hbm.at[wid], hi_v)

        def body(sb, carry):
            copies = []
            for bq in range(_KFIRE):
                j = sb * _KFIRE + bq
                copies.append(pltpu.async_copy(
                    flat_hbm.at[lo_v.at[j]],
                    buf0_v.at[pl.ds(bq * chunk, chunk)], gsem))
                copies.append(pltpu.async_copy(
                    flat_hbm.at[hi_v.at[j]],
                    buf1_v.at[pl.ds(bq * chunk, chunk)], gsem))
            for c in copies:
                c.wait()
            base = (wid * n_super + sb) * sup
            pltpu.sync_copy(buf0_v, out0_hbm.at[pl.ds(base, sup)])
            pltpu.sync_copy(buf1_v, out1_hbm.at[pl.ds(base, sup)])
            return carry

        lax.fori_loop(0, n_super, body, 0)

    return k(flat, idx_lo, idx_hi)


def kernel(x, table, W, b):
    bsz, fields = x.shape
    v = table.shape[0]
    n = bsz * fields
    n_chunks = n // (_NW * _CHUNK)
    idx_lo = x.reshape(_NW, n_chunks, _CHUNK)
    idx_hi = idx_lo + v

    w = 0.5 * W
    b2 = (b + 0.5 * W.sum(axis=1)).reshape(2, 1)

    planes = _tc_head_table(table, w, b2)
    y0, y1 = _sc_lookup(planes.reshape(2 * v), idx_lo, idx_hi)
    out = jnp.stack([y0, y1], axis=-1)
    return out.reshape(bsz, fields, 2)


# speedup vs baseline: 1.7051x; 1.0119x over previous
import functools

import jax
import jax.numpy as jnp
from jax import lax
from jax.experimental import pallas as pl
from jax.experimental.pallas import tpu as pltpu
from jax.experimental.pallas import tpu_sc as plsc

_H = 64
_NC = 2
_NS = 16
_NW = _NC * _NS
_CHUNK = 128
_KFIRE = 8


def _tc_head_table(table, w, b2):
    v = table.shape[0]
    blk = 16384
    nfull = v // blk
    tail = v - nfull * blk
    v_pad = (nfull + 1) * blk
    nslot = 4

    def body(t_hbm, w_ref, b_ref, o_hbm, ibuf, obuf, isem, osem):
        def in_copy(i, slot):
            return pltpu.make_async_copy(
                t_hbm.at[pl.ds(i * blk, blk), :], ibuf.at[slot],
                isem.at[slot])

        tail_in = pltpu.make_async_copy(
            t_hbm.at[pl.ds(nfull * blk, tail), :],
            ibuf.at[nslot, pl.ds(0, tail)], isem.at[nslot])

        for p in range(nslot):
            in_copy(p, p).start()
        tail_in.start()

        def head(s_buf):
            s = jnp.tanh(0.5 * s_buf)
            y = lax.dot_general(
                w_ref[...], s, (((1,), (1,)), ((), ())),
                preferred_element_type=jnp.float32,
            )
            return y + b_ref[...]

        def step(i, carry):
            slot = lax.rem(i, nslot)
            oslot = lax.rem(i, 2)
            in_copy(i, slot).wait()

            @pl.when(i >= 2)
            def _():
                pltpu.make_async_copy(
                    obuf.at[oslot], o_hbm.at[:, pl.ds((i - 2) * blk, blk)],
                    osem.at[oslot]).wait()

            obuf[oslot] = head(ibuf[slot])
            pltpu.make_async_copy(
                obuf.at[oslot], o_hbm.at[:, pl.ds(i * blk, blk)],
                osem.at[oslot]).start()

            @pl.when(i + nslot < nfull)
            def _():
                in_copy(i + nslot, slot).start()

            return carry

        lax.fori_loop(0, nfull, step, 0)

        toslot = nfull % 2
        pltpu.make_async_copy(
            obuf.at[toslot], o_hbm.at[:, pl.ds((nfull - 2) * blk, blk)],
            osem.at[toslot]).wait()
        tail_in.wait()
        obuf[toslot] = head(ibuf[nslot])
        tail_out = pltpu.make_async_copy(
            obuf.at[toslot], o_hbm.at[:, pl.ds(nfull * blk, blk)],
            osem.at[toslot])
        tail_out.start()
        pltpu.make_async_copy(
            obuf.at[1 - toslot], o_hbm.at[:, pl.ds((nfull - 1) * blk, blk)],
            osem.at[1 - toslot]).wait()
        tail_out.wait()

    return pl.pallas_call(
        body,
        in_specs=[
            pl.BlockSpec(memory_space=pl.ANY),
            pl.BlockSpec(memory_space=pltpu.MemorySpace.VMEM),
            pl.BlockSpec(memory_space=pltpu.MemorySpace.VMEM),
        ],
        out_specs=pl.BlockSpec(memory_space=pl.ANY),
        out_shape=jax.ShapeDtypeStruct((2, v_pad), jnp.float32),
        scratch_shapes=[
            pltpu.VMEM((nslot + 1, blk, _H), jnp.float32),
            pltpu.VMEM((2, 2, blk), jnp.float32),
            pltpu.SemaphoreType.DMA((nslot + 1,)),
            pltpu.SemaphoreType.DMA((2,)),
        ],
    )(table, w, b2)


def _sc_lookup(flat, idx_lo, idx_hi):
    nw, n_chunks, chunk = idx_lo.shape
    n = nw * n_chunks * chunk
    n_super = n_chunks // _KFIRE
    sup = _KFIRE * chunk
    mesh = plsc.VectorSubcoreMesh(core_axis_name="c", subcore_axis_name="s")

    @functools.partial(
        pl.kernel,
        out_type=(
            jax.ShapeDtypeStruct((n,), jnp.float32),
            jax.ShapeDtypeStruct((n,), jnp.float32),
        ),
        mesh=mesh,
        compiler_params=pltpu.CompilerParams(use_tc_tiling_on_sc=False),
        scratch_types=[
            pltpu.VMEM((n_chunks, chunk), jnp.int32),
            pltpu.VMEM((n_chunks, chunk), jnp.int32),
            pltpu.VMEM((sup,), jnp.float32),
            pltpu.VMEM((sup,), jnp.float32),
            pltpu.SemaphoreType.DMA,
        ],
    )
    def k(flat_hbm, lo_hbm, hi_hbm, out0_hbm, out1_hbm,
          lo_v, hi_v, buf0_v, buf1_v, gsem):
        wid = lax.axis_index("s") * _NC + lax.axis_index("c")
        pltpu.sync_copy(lo_hbm.at[wid], lo_v)
        pltpu.sync_copy(hi_hbm.at[wid], hi_v)

        def body(sb, carry):
            copies = []
            for bq in range(_KFIRE):
                j = sb * _KFIRE + bq
                copies.append(pltpu.async_copy(
                    flat_hbm.at[lo_v.at[j]],
                    buf0_v.at[pl.ds(bq * chunk, chunk)], gsem))
                copies.append(pltpu.async_copy(
                    flat_hbm.at[hi_v.at[j]],
                    buf1_v.at[pl.ds(bq * chunk, chunk)], gsem))
            for c in copies:
                c.wait()
            base = (wid * n_super + sb) * sup
            pltpu.sync_copy(buf0_v, out0_hbm.at[pl.ds(base, sup)])
            pltpu.sync_copy(buf1_v, out1_hbm.at[pl.ds(base, sup)])
            return carry

        lax.fori_loop(0, n_super, body, 0)

    return k(flat, idx_lo, idx_hi)


def kernel(x, table, W, b):
    bsz, fields = x.shape
    v = table.shape[0]
    n = bsz * fields
    n_chunks = n // (_NW * _CHUNK)
    idx_lo = x.reshape(_NW, n_chunks, _CHUNK)

    w = 0.5 * W
    b2 = (b + 0.5 * W.sum(axis=1)).reshape(2, 1)

    planes = _tc_head_table(table, w, b2)
    v_pad = planes.shape[1]
    idx_hi = idx_lo + v_pad
    y0, y1 = _sc_lookup(planes.reshape(2 * v_pad), idx_lo, idx_hi)
    out = jnp.stack([y0, y1], axis=-1)
    return out.reshape(bsz, fields, 2)


# speedup vs baseline: 1.7114x; 1.0037x over previous
import functools

import jax
import jax.numpy as jnp
from jax import lax
from jax.experimental import pallas as pl
from jax.experimental.pallas import tpu as pltpu
from jax.experimental.pallas import tpu_sc as plsc

_H = 64
_NC = 2
_NS = 16
_NW = _NC * _NS
_CHUNK = 128
_KFIRE = 13


def _tc_head_table(table, w, b2):
    v = table.shape[0]
    blk = 16384
    nfull = v // blk
    tail = v - nfull * blk
    v_pad = (nfull + 1) * blk
    nslot = 4

    def body(t_hbm, w_ref, b_ref, o_hbm, ibuf, obuf, isem, osem):
        def in_copy(i, slot):
            return pltpu.make_async_copy(
                t_hbm.at[pl.ds(i * blk, blk), :], ibuf.at[slot],
                isem.at[slot])

        tail_in = pltpu.make_async_copy(
            t_hbm.at[pl.ds(nfull * blk, tail), :],
            ibuf.at[nslot, pl.ds(0, tail)], isem.at[nslot])

        for p in range(nslot):
            in_copy(p, p).start()
        tail_in.start()

        def head(s_buf):
            s = jnp.tanh(0.5 * s_buf)
            y = lax.dot_general(
                w_ref[...], s, (((1,), (1,)), ((), ())),
                preferred_element_type=jnp.float32,
            )
            return y + b_ref[...]

        def step(i, carry):
            slot = lax.rem(i, nslot)
            oslot = lax.rem(i, 2)
            in_copy(i, slot).wait()

            @pl.when(i >= 2)
            def _():
                pltpu.make_async_copy(
                    obuf.at[oslot], o_hbm.at[:, pl.ds((i - 2) * blk, blk)],
                    osem.at[oslot]).wait()

            obuf[oslot] = head(ibuf[slot])
            pltpu.make_async_copy(
                obuf.at[oslot], o_hbm.at[:, pl.ds(i * blk, blk)],
                osem.at[oslot]).start()

            @pl.when(i + nslot < nfull)
            def _():
                in_copy(i + nslot, slot).start()

            return carry

        lax.fori_loop(0, nfull, step, 0)

        toslot = nfull % 2
        pltpu.make_async_copy(
            obuf.at[toslot], o_hbm.at[:, pl.ds((nfull - 2) * blk, blk)],
            osem.at[toslot]).wait()
        tail_in.wait()
        obuf[toslot] = head(ibuf[nslot])
        tail_out = pltpu.make_async_copy(
            obuf.at[toslot], o_hbm.at[:, pl.ds(nfull * blk, blk)],
            osem.at[toslot])
        tail_out.start()
        pltpu.make_async_copy(
            obuf.at[1 - toslot], o_hbm.at[:, pl.ds((nfull - 1) * blk, blk)],
            osem.at[1 - toslot]).wait()
        tail_out.wait()

    return pl.pallas_call(
        body,
        in_specs=[
            pl.BlockSpec(memory_space=pl.ANY),
            pl.BlockSpec(memory_space=pltpu.MemorySpace.VMEM),
            pl.BlockSpec(memory_space=pltpu.MemorySpace.VMEM),
        ],
        out_specs=pl.BlockSpec(memory_space=pl.ANY),
        out_shape=jax.ShapeDtypeStruct((2, v_pad), jnp.float32),
        scratch_shapes=[
            pltpu.VMEM((nslot + 1, blk, _H), jnp.float32),
            pltpu.VMEM((2, 2, blk), jnp.float32),
            pltpu.SemaphoreType.DMA((nslot + 1,)),
            pltpu.SemaphoreType.DMA((2,)),
        ],
    )(table, w, b2)


def _sc_lookup(flat, idx_lo, idx_hi):
    nw, n_chunks, chunk = idx_lo.shape
    n = nw * n_chunks * chunk
    n_super = n_chunks // _KFIRE
    sup = _KFIRE * chunk
    mesh = plsc.VectorSubcoreMesh(core_axis_name="c", subcore_axis_name="s")

    @functools.partial(
        pl.kernel,
        out_type=(
            jax.ShapeDtypeStruct((n,), jnp.float32),
            jax.ShapeDtypeStruct((n,), jnp.float32),
        ),
        mesh=mesh,
        compiler_params=pltpu.CompilerParams(use_tc_tiling_on_sc=False),
        scratch_types=[
            pltpu.VMEM((n_chunks, chunk), jnp.int32),
            pltpu.VMEM((n_chunks, chunk), jnp.int32),
            pltpu.VMEM((sup,), jnp.float32),
            pltpu.VMEM((sup,), jnp.float32),
            pltpu.SemaphoreType.DMA,
        ],
    )
    def k(flat_hbm, lo_hbm, hi_hbm, out0_hbm, out1_hbm,
          lo_v, hi_v, buf0_v, buf1_v, gsem):
        wid = lax.axis_index("s") * _NC + lax.axis_index("c")
        pltpu.sync_copy(lo_hbm.at[wid], lo_v)
        pltpu.sync_copy(hi_hbm.at[wid], hi_v)

        def body(sb, carry):
            copies = []
            for bq in range(_KFIRE):
                j = sb * _KFIRE + bq
                copies.append(pltpu.async_copy(
                    flat_hbm.at[lo_v.at[j]],
                    buf0_v.at[pl.ds(bq * chunk, chunk)], gsem))
                copies.append(pltpu.async_copy(
                    flat_hbm.at[hi_v.at[j]],
                    buf1_v.at[pl.ds(bq * chunk, chunk)], gsem))
            for c in copies:
                c.wait()
            base = (wid * n_super + sb) * sup
            pltpu.sync_copy(buf0_v, out0_hbm.at[pl.ds(base, sup)])
            pltpu.sync_copy(buf1_v, out1_hbm.at[pl.ds(base, sup)])
            return carry

        lax.fori_loop(0, n_super, body, 0)

    return k(flat, idx_lo, idx_hi)


def kernel(x, table, W, b):
    bsz, fields = x.shape
    v = table.shape[0]
    n = bsz * fields
    n_chunks = n // (_NW * _CHUNK)
    idx_lo = x.reshape(_NW, n_chunks, _CHUNK)

    w = 0.5 * W
    b2 = (b + 0.5 * W.sum(axis=1)).reshape(2, 1)

    planes = _tc_head_table(table, w, b2)
    v_pad = planes.shape[1]
    idx_hi = idx_lo + v_pad
    y0, y1 = _sc_lookup(planes.reshape(2 * v_pad), idx_lo, idx_hi)
    out = jnp.stack([y0, y1], axis=-1)
    return out.reshape(bsz, fields, 2)


# speedup vs baseline: 1.7293x; 1.0105x over previous
import functools

import jax
import jax.numpy as jnp
from jax import lax
from jax.experimental import pallas as pl
from jax.experimental.pallas import tpu as pltpu
from jax.experimental.pallas import tpu_sc as plsc

_H = 64
_NC = 2
_NS = 16
_NW = _NC * _NS
_CHUNK = 128
_KFIRE = 13


def _tc_head_table(table, w, b2):
    v = table.shape[0]
    blk = 16384
    nfull = v // blk
    tail = v - nfull * blk
    v_pad = (nfull + 1) * blk
    nslot = 4

    def body(t_hbm, w_ref, b_ref, o_hbm, ibuf, obuf, isem, osem):
        def in_copy(i, slot):
            return pltpu.make_async_copy(
                t_hbm.at[pl.ds(i * blk, blk), :], ibuf.at[slot],
                isem.at[slot])

        def out_copies(i, oslot):
            return (
                pltpu.make_async_copy(
                    obuf.at[oslot, 0], o_hbm.at[pl.ds(i * blk, blk)],
                    osem.at[oslot]),
                pltpu.make_async_copy(
                    obuf.at[oslot, 1],
                    o_hbm.at[pl.ds(v_pad + i * blk, blk)], osem.at[oslot]),
            )

        tail_in = pltpu.make_async_copy(
            t_hbm.at[pl.ds(nfull * blk, tail), :],
            ibuf.at[nslot, pl.ds(0, tail)], isem.at[nslot])

        for p in range(nslot):
            in_copy(p, p).start()
        tail_in.start()

        def head(s_buf):
            s = jnp.tanh(0.5 * s_buf)
            y = lax.dot_general(
                w_ref[...], s, (((1,), (1,)), ((), ())),
                preferred_element_type=jnp.float32,
            )
            return y + b_ref[...]

        def step(i, carry):
            slot = lax.rem(i, nslot)
            oslot = lax.rem(i, 2)
            in_copy(i, slot).wait()

            @pl.when(i >= 2)
            def _():
                for c in out_copies(i - 2, oslot):
                    c.wait()

            obuf[oslot] = head(ibuf[slot])
            for c in out_copies(i, oslot):
                c.start()

            @pl.when(i + nslot < nfull)
            def _():
                in_copy(i + nslot, slot).start()

            return carry

        lax.fori_loop(0, nfull, step, 0)

        toslot = nfull % 2
        for c in out_copies(nfull - 2, toslot):
            c.wait()
        tail_in.wait()
        obuf[toslot] = head(ibuf[nslot])
        tail_out = out_copies(nfull, toslot)
        for c in tail_out:
            c.start()
        for c in out_copies(nfull - 1, 1 - toslot):
            c.wait()
        for c in tail_out:
            c.wait()

    return pl.pallas_call(
        body,
        in_specs=[
            pl.BlockSpec(memory_space=pl.ANY),
            pl.BlockSpec(memory_space=pltpu.MemorySpace.VMEM),
            pl.BlockSpec(memory_space=pltpu.MemorySpace.VMEM),
        ],
        out_specs=pl.BlockSpec(memory_space=pl.ANY),
        out_shape=jax.ShapeDtypeStruct((2 * v_pad,), jnp.float32),
        scratch_shapes=[
            pltpu.VMEM((nslot + 1, blk, _H), jnp.float32),
            pltpu.VMEM((2, 2, blk), jnp.float32),
            pltpu.SemaphoreType.DMA((nslot + 1,)),
            pltpu.SemaphoreType.DMA((2,)),
        ],
    )(table, w, b2)


def _sc_lookup(flat, idx_lo, idx_hi):
    nw, n_chunks, chunk = idx_lo.shape
    n = nw * n_chunks * chunk
    n_super = n_chunks // _KFIRE
    sup = _KFIRE * chunk
    mesh = plsc.VectorSubcoreMesh(core_axis_name="c", subcore_axis_name="s")

    @functools.partial(
        pl.kernel,
        out_type=(
            jax.ShapeDtypeStruct((n,), jnp.float32),
            jax.ShapeDtypeStruct((n,), jnp.float32),
        ),
        mesh=mesh,
        compiler_params=pltpu.CompilerParams(use_tc_tiling_on_sc=False),
        scratch_types=[
            pltpu.VMEM((n_chunks, chunk), jnp.int32),
            pltpu.VMEM((n_chunks, chunk), jnp.int32),
            pltpu.VMEM((sup,), jnp.float32),
            pltpu.VMEM((sup,), jnp.float32),
            pltpu.SemaphoreType.DMA,
        ],
    )
    def k(flat_hbm, lo_hbm, hi_hbm, out0_hbm, out1_hbm,
          lo_v, hi_v, buf0_v, buf1_v, gsem):
        wid = lax.axis_index("s") * _NC + lax.axis_index("c")
        pltpu.sync_copy(lo_hbm.at[wid], lo_v)
        pltpu.sync_copy(hi_hbm.at[wid], hi_v)

        def body(sb, carry):
            copies = []
            for bq in range(_KFIRE):
                j = sb * _KFIRE + bq
                copies.append(pltpu.async_copy(
                    flat_hbm.at[lo_v.at[j]],
                    buf0_v.at[pl.ds(bq * chunk, chunk)], gsem))
                copies.append(pltpu.async_copy(
                    flat_hbm.at[hi_v.at[j]],
                    buf1_v.at[pl.ds(bq * chunk, chunk)], gsem))
            for c in copies:
                c.wait()
            base = (wid * n_super + sb) * sup
            pltpu.sync_copy(buf0_v, out0_hbm.at[pl.ds(base, sup)])
            pltpu.sync_copy(buf1_v, out1_hbm.at[pl.ds(base, sup)])
            return carry

        lax.fori_loop(0, n_super, body, 0)

    return k(flat, idx_lo, idx_hi)


def kernel(x, table, W, b):
    bsz, fields = x.shape
    v = table.shape[0]
    n = bsz * fields
    n_chunks = n // (_NW * _CHUNK)
    idx_lo = x.reshape(_NW, n_chunks, _CHUNK)

    w = 0.5 * W
    b2 = (b + 0.5 * W.sum(axis=1)).reshape(2, 1)

    flat = _tc_head_table(table, w, b2)
    v_pad = flat.shape[0] // 2
    idx_hi = idx_lo + v_pad
    y0, y1 = _sc_lookup(flat, idx_lo, idx_hi)
    out = jnp.stack([y0, y1], axis=-1)
    return out.reshape(bsz, fields, 2)
